# baseline (speedup 1.0000x reference)
import jax
import jax.numpy as jnp
from jax import lax
from jax.experimental import pallas as pl
from jax.experimental.pallas import tpu as pltpu
from jax.experimental.pallas import tpu_sc as plsc

BATCH = 16384
HIST = 200
OUT_H = 199
D = 32
C1 = 128
C2 = 72

_info = plsc.get_sparse_core_info()
_NC = _info.num_cores
_NS = _info.num_subcores
_NW = _NC * _NS
_ROWS_PER_W = BATCH // _NW
_RB = 8
_NBLK = _ROWS_PER_W // _RB


def _gather_copies(table_hbm, idx_v, rows_v, sem, buf):
    for r in range(_RB):
        yield pltpu.make_async_copy(
            table_hbm.at[idx_v.at[buf, r, pl.ds(0, C1)]],
            rows_v.at[buf, r, pl.ds(0, C1)], sem)
        yield pltpu.make_async_copy(
            table_hbm.at[idx_v.at[buf, r, pl.ds(C1, C2)]],
            rows_v.at[buf, r, pl.ds(C1, C2)], sem)


def _embed_body(idx_hbm, table_hbm, out_hbm,
                idx_v, rows_v, sem_a, sem_b, sem_out):
    wid = lax.axis_index("s") * _NC + lax.axis_index("c")
    base = wid * _ROWS_PER_W
    sems = (sem_a, sem_b)

    def stage_fire(g, buf):
        rbase = base + g * _RB
        pltpu.sync_copy(idx_hbm.at[pl.ds(rbase, _RB)], idx_v.at[buf])
        for cp in _gather_copies(table_hbm, idx_v, rows_v, sems[buf], buf):
            cp.start()

    def drain_gathers(buf):
        for cp in _gather_copies(table_hbm, idx_v, rows_v, sems[buf], buf):
            cp.wait()

    def out_copies(g, buf):
        rbase = base + g * _RB
        for r in range(_RB):
            yield pltpu.make_async_copy(
                rows_v.at[buf, r, pl.ds(0, OUT_H)],
                out_hbm.at[pl.ds((rbase + r) * OUT_H, OUT_H)],
                sem_out)

    stage_fire(0, 0)

    def outer(gg, c):
        g0 = 2 * gg
        for j in range(2):
            g = g0 + j
            nxt = 1 - j

            @pl.when(g + 1 < _NBLK)
            def _():
                @pl.when(g >= 1)
                def _():
                    for cp in out_copies(g - 1, nxt):
                        cp.wait()
                stage_fire(g + 1, nxt)

            drain_gathers(j)
            for cp in out_copies(g, j):
                cp.start()
        return c

    lax.fori_loop(0, _NBLK // 2, outer, 0)
    for cp in out_copies(_NBLK - 2, 0):
        cp.wait()
    for cp in out_copies(_NBLK - 1, 1):
        cp.wait()


def kernel(order_feat, embed_table):
    k = pl.kernel(
        _embed_body,
        out_type=jax.ShapeDtypeStruct((BATCH * OUT_H, D), jnp.float32),
        mesh=plsc.VectorSubcoreMesh(core_axis_name="c", subcore_axis_name="s"),
        scratch_types=[
            pltpu.VMEM((2, _RB, HIST), jnp.int32),
            pltpu.VMEM((2, _RB, HIST, D), jnp.float32),
            pltpu.SemaphoreType.DMA,
            pltpu.SemaphoreType.DMA,
            pltpu.SemaphoreType.DMA,
        ],
        compiler_params=pltpu.CompilerParams(use_tc_tiling_on_sc=False),
    )
    return k(order_feat, embed_table).reshape(BATCH, OUT_H, D)


# speedup vs baseline: 3.5579x; 3.5579x over previous
import jax
import jax.numpy as jnp
from jax import lax
from jax.experimental import pallas as pl
from jax.experimental.pallas import tpu as pltpu
from jax.experimental.pallas import tpu_sc as plsc

BATCH = 16384
HIST = 200
OUT_H = 199
D = 32
FG = 4
FI = 8
BI = 128
L = 16

_info = plsc.get_sparse_core_info()
_NC = _info.num_cores
_NS = _info.num_subcores
_NW = _NC * _NS
_NBG = BATCH // BI
_BG_PER_W = _NBG // _NW


def _embed_body(idx_hbm, table_hbm, out_hbm,
                idx_v, idx_col, rows_v, tile_v, sem_g0, sem_g1,
                sem_o0, sem_o1, sem_i):
    wid = lax.axis_index("s") * _NC + lax.axis_index("c")
    sem_g = (sem_g0, sem_g1)
    sem_o = (sem_o0, sem_o1)
    lane = jnp.arange(L, dtype=jnp.int32)

    def build_idx_col(t, buf):
        tcol = jnp.full((L,), t, dtype=jnp.int32)
        for j in range(BI // L):
            v = plsc.load_gather(idx_v, [lane + (j * L), tcol])
            idx_col[buf, pl.ds(j * L, L)] = v

    def gather_copy(buf):
        return pltpu.make_async_copy(
            table_hbm.at[idx_col.at[buf]], rows_v.at[buf], sem_g[buf])

    def transpose_block(buf):
        for fg in range(FG):
            for fi in range(FI):
                f = jnp.full((L,), fg * FI + fi, dtype=jnp.int32)
                for j in range(BI // L):
                    v = plsc.load_gather(rows_v.at[buf],
                                         [lane + (j * L), f])
                    tile_v[buf, fg, fi, pl.ds(j * L, L)] = v

    def out_copy(t, bg, buf):
        return pltpu.make_async_copy(
            tile_v.at[buf], out_hbm.at[t, pl.ds(0, FG), bg], sem_o[buf])

    def per_bg(bgi, carry):
        bg = wid * _BG_PER_W + bgi
        pltpu.make_async_copy(
            idx_hbm.at[pl.ds(bg * BI, BI)], idx_v, sem_i).start()
        pltpu.make_async_copy(
            idx_hbm.at[pl.ds(bg * BI, BI)], idx_v, sem_i).wait()
        build_idx_col(0, 0)
        gather_copy(0).start()

        def per_t2(gg, c):
            for j01 in range(2):
                t = 2 * gg + j01

                @pl.when(t < OUT_H)
                def _():
                    @pl.when(t + 1 < OUT_H)
                    def _():
                        build_idx_col(t + 1, 1 - j01)
                        gather_copy(1 - j01).start()
                    gather_copy(j01).wait()

                    @pl.when(t >= 2)
                    def _():
                        out_copy(t - 2, bg, j01).wait()
                    transpose_block(j01)
                    out_copy(t, bg, j01).start()
            return c

        lax.fori_loop(0, (OUT_H + 2) // 2, per_t2, 0)
        out_copy(OUT_H - 2, bg, 1).wait()
        out_copy(OUT_H - 1, bg, 0).wait()
        return carry

    lax.fori_loop(0, _BG_PER_W, per_bg, 0)


def kernel(order_feat, embed_table):
    k = pl.kernel(
        _embed_body,
        out_type=jax.ShapeDtypeStruct((OUT_H, FG, _NBG, FI, BI),
                                      jnp.float32),
        mesh=plsc.VectorSubcoreMesh(core_axis_name="c", subcore_axis_name="s"),
        scratch_types=[
            pltpu.VMEM((BI, HIST), jnp.int32),
            pltpu.VMEM((2, BI), jnp.int32),
            pltpu.VMEM((2, BI, D), jnp.float32),
            pltpu.VMEM((2, FG, FI, BI), jnp.float32),
            pltpu.SemaphoreType.DMA,
            pltpu.SemaphoreType.DMA,
            pltpu.SemaphoreType.DMA,
            pltpu.SemaphoreType.DMA,
            pltpu.SemaphoreType.DMA,
        ],
        compiler_params=pltpu.CompilerParams(use_tc_tiling_on_sc=False,
                                             needs_layout_passes=False),
    )
    out5 = k(order_feat, embed_table)
    return out5.transpose(2, 4, 0, 1, 3).reshape(BATCH, OUT_H, D)


# speedup vs baseline: 4.6214x; 1.2989x over previous
import jax
import jax.numpy as jnp
from jax import lax
from jax.experimental import pallas as pl
from jax.experimental.pallas import tpu as pltpu
from jax.experimental.pallas import tpu_sc as plsc

BATCH = 16384
HIST = 200
OUT_H = 199
D = 32
FG = 4
FI = 8
BI = 128
L = 16

_info = plsc.get_sparse_core_info()
_NC = _info.num_cores
_NS = _info.num_subcores
_NW = _NC * _NS
_NBG = BATCH // BI
_BG_PER_W = _NBG // _NW


def _embed_body(idx_hbm, table_hbm, out_hbm,
                idx_v, idx_col, rows_v, tile_v, sem_g0, sem_g1,
                sem_o0, sem_o1, sem_i):
    wid = lax.axis_index("s") * _NC + lax.axis_index("c")
    sem_g = (sem_g0, sem_g1)
    sem_o = (sem_o0, sem_o1)
    lane = jnp.arange(L, dtype=jnp.int32)

    def build_idx_col(t, buf):
        tcol = jnp.full((L,), t, dtype=jnp.int32)
        for j in range(BI // L):
            v = plsc.load_gather(idx_v, [lane + (j * L), tcol])
            idx_col[buf, pl.ds(j * L, L)] = v

    def gather_copy(buf):
        return pltpu.make_async_copy(
            table_hbm.at[idx_col.at[buf]], rows_v.at[buf], sem_g[buf])

    def transpose_block(buf):
        @plsc.parallel_loop(0, FG * FI * (BI // L), unroll=8)
        def _(i):
            f = i >> 3
            b16 = i & 7
            v = plsc.load_gather(
                rows_v.at[buf],
                [lane + b16 * L, jnp.full((L,), 0, jnp.int32) + f])
            tile_v[buf, pl.ds(i * L, L)] = v

    def out_copies(t, bg, buf):
        for fg in range(FG):
            off = ((t * FG + fg) * _NBG + bg) * (FI * BI)
            yield pltpu.make_async_copy(
                tile_v.at[buf, pl.ds(fg * FI * BI, FI * BI)],
                out_hbm.at[pl.ds(off, FI * BI)], sem_o[buf])

    def per_bg(bgi, carry):
        bg = wid * _BG_PER_W + bgi
        pltpu.make_async_copy(
            idx_hbm.at[pl.ds(bg * BI, BI)], idx_v, sem_i).start()
        pltpu.make_async_copy(
            idx_hbm.at[pl.ds(bg * BI, BI)], idx_v, sem_i).wait()
        build_idx_col(0, 0)
        gather_copy(0).start()

        def per_t2(gg, c):
            for j01 in range(2):
                t = 2 * gg + j01

                @pl.when(t < OUT_H)
                def _():
                    @pl.when(t + 1 < OUT_H)
                    def _():
                        build_idx_col(t + 1, 1 - j01)
                        gather_copy(1 - j01).start()
                    gather_copy(j01).wait()

                    @pl.when(t >= 2)
                    def _():
                        for cp in out_copies(t - 2, bg, j01):
                            cp.wait()
                    transpose_block(j01)
                    for cp in out_copies(t, bg, j01):
                        cp.start()
            return c

        lax.fori_loop(0, (OUT_H + 2) // 2, per_t2, 0)
        for cp in out_copies(OUT_H - 2, bg, 1):
            cp.wait()
        for cp in out_copies(OUT_H - 1, bg, 0):
            cp.wait()
        return carry

    lax.fori_loop(0, _BG_PER_W, per_bg, 0)


def kernel(order_feat, embed_table):
    k = pl.kernel(
        _embed_body,
        out_type=jax.ShapeDtypeStruct((OUT_H * FG * _NBG * FI * BI,),
                                      jnp.float32),
        mesh=plsc.VectorSubcoreMesh(core_axis_name="c", subcore_axis_name="s"),
        scratch_types=[
            pltpu.VMEM((BI, HIST), jnp.int32),
            pltpu.VMEM((2, BI), jnp.int32),
            pltpu.VMEM((2, BI, D), jnp.float32),
            pltpu.VMEM((2, FG * FI * BI), jnp.float32),
            pltpu.SemaphoreType.DMA,
            pltpu.SemaphoreType.DMA,
            pltpu.SemaphoreType.DMA,
            pltpu.SemaphoreType.DMA,
            pltpu.SemaphoreType.DMA,
        ],
        compiler_params=pltpu.CompilerParams(use_tc_tiling_on_sc=False,
                                             needs_layout_passes=False),
    )
    out5 = k(order_feat, embed_table).reshape(OUT_H, FG, _NBG, FI, BI)
    return out5.transpose(2, 4, 0, 1, 3).reshape(BATCH, OUT_H, D)


# speedup vs baseline: 5.6020x; 1.2122x over previous
import jax
import jax.numpy as jnp
from jax import lax
from jax.experimental import pallas as pl
from jax.experimental.pallas import tpu as pltpu
from jax.experimental.pallas import tpu_sc as plsc

BATCH = 16384
HIST = 200
OUT_H = 199
D = 32
FG = 4
FI = 8
BI = 128
L = 16

_info = plsc.get_sparse_core_info()
_NC = _info.num_cores
_NS = _info.num_subcores
_NW = _NC * _NS
_NBG = BATCH // BI
_BG_PER_W = _NBG // _NW


def _embed_body(idx_hbm, table_hbm, out_hbm,
                idx_v, idx_col, rows_v, tile_v, sem_g0, sem_g1,
                sem_o0, sem_o1, sem_i):
    wid = lax.axis_index("s") * _NC + lax.axis_index("c")
    sem_g = (sem_g0, sem_g1)
    sem_o = (sem_o0, sem_o1)
    lane = jnp.arange(L, dtype=jnp.int32)

    def build_idx_col(t, buf):
        tcol = jnp.full((L,), t, dtype=jnp.int32)
        for j in range(BI // L):
            v = plsc.load_gather(idx_v, [lane + (j * L), tcol])
            idx_col[buf, pl.ds(j * L, L)] = v

    def gather_copy(buf):
        return pltpu.make_async_copy(
            table_hbm.at[idx_col.at[buf]], rows_v.at[buf], sem_g[buf])

    lane128 = lane * BI

    def transpose_block(buf):
        @plsc.parallel_loop(0, BI, unroll=8)
        def _(r):
            iv0 = lane128 + r
            v0 = rows_v[buf, r, pl.ds(0, L)]
            plsc.store_scatter(tile_v.at[buf], [iv0], v0)
            iv1 = iv0 + (L * BI)
            v1 = rows_v[buf, r, pl.ds(L, L)]
            plsc.store_scatter(tile_v.at[buf], [iv1], v1)

    def out_copies(t, bg, buf):
        for fg in range(FG):
            off = ((t * FG + fg) * _NBG + bg) * (FI * BI)
            yield pltpu.make_async_copy(
                tile_v.at[buf, pl.ds(fg * FI * BI, FI * BI)],
                out_hbm.at[pl.ds(off, FI * BI)], sem_o[buf])

    def per_bg(bgi, carry):
        bg = wid * _BG_PER_W + bgi
        pltpu.make_async_copy(
            idx_hbm.at[pl.ds(bg * BI, BI)], idx_v, sem_i).start()
        pltpu.make_async_copy(
            idx_hbm.at[pl.ds(bg * BI, BI)], idx_v, sem_i).wait()
        build_idx_col(0, 0)
        gather_copy(0).start()

        def per_t2(gg, c):
            for j01 in range(2):
                t = 2 * gg + j01

                @pl.when(t < OUT_H)
                def _():
                    @pl.when(t + 1 < OUT_H)
                    def _():
                        build_idx_col(t + 1, 1 - j01)
                        gather_copy(1 - j01).start()
                    gather_copy(j01).wait()

                    @pl.when(t >= 2)
                    def _():
                        for cp in out_copies(t - 2, bg, j01):
                            cp.wait()
                    transpose_block(j01)
                    for cp in out_copies(t, bg, j01):
                        cp.start()
            return c

        lax.fori_loop(0, (OUT_H + 2) // 2, per_t2, 0)
        for cp in out_copies(OUT_H - 2, bg, 1):
            cp.wait()
        for cp in out_copies(OUT_H - 1, bg, 0):
            cp.wait()
        return carry

    lax.fori_loop(0, _BG_PER_W, per_bg, 0)


def kernel(order_feat, embed_table):
    k = pl.kernel(
        _embed_body,
        out_type=jax.ShapeDtypeStruct((OUT_H * FG * _NBG * FI * BI,),
                                      jnp.float32),
        mesh=plsc.VectorSubcoreMesh(core_axis_name="c", subcore_axis_name="s"),
        scratch_types=[
            pltpu.VMEM((BI, HIST), jnp.int32),
            pltpu.VMEM((2, BI), jnp.int32),
            pltpu.VMEM((2, BI, D), jnp.float32),
            pltpu.VMEM((2, FG * FI * BI), jnp.float32),
            pltpu.SemaphoreType.DMA,
            pltpu.SemaphoreType.DMA,
            pltpu.SemaphoreType.DMA,
            pltpu.SemaphoreType.DMA,
            pltpu.SemaphoreType.DMA,
        ],
        compiler_params=pltpu.CompilerParams(use_tc_tiling_on_sc=False,
                                             needs_layout_passes=False),
    )
    out5 = k(order_feat, embed_table).reshape(OUT_H, FG, _NBG, FI, BI)
    return out5.transpose(2, 4, 0, 1, 3).reshape(BATCH, OUT_H, D)
